# baseline (speedup 1.0000x reference)
import functools

import jax
import jax.numpy as jnp
from jax import lax
from jax.experimental import pallas as pl
from jax.experimental.pallas import tpu as pltpu
from jax.experimental.pallas import tpu_sc as plsc

NUM_USERS = 1000000
NUM_PRODUCTS = 1000000
EMBED = 64
BATCH = 16384

NC, NS, LANES = 2, 16, 16
NW = NC * NS
ROWS_PER_W = BATCH // NW
GATHER_CHUNK = 128
NCHUNK = ROWS_PER_W // GATHER_CHUNK


def _sc_kernel(user_ids2d, product_ids2d, user_table, product_table,
               wu, wp, bvec, out_hbm,
               idx_u, idx_p, u_rows, p_rows, out_v, wu_v, wp_v, b_v, sem):
    wid = lax.axis_index("s") * NC + lax.axis_index("c")
    base_chunk = wid * NCHUNK

    pltpu.sync_copy(user_ids2d.at[pl.ds(base_chunk, NCHUNK)], idx_u)
    pltpu.sync_copy(product_ids2d.at[pl.ds(base_chunk, NCHUNK)], idx_p)
    pltpu.sync_copy(wu, wu_v)
    pltpu.sync_copy(wp, wp_v)
    pltpu.sync_copy(bvec, b_v)

    copies = []
    for k in range(NCHUNK):
        copies.append(pltpu.async_copy(
            user_table.at[idx_u.at[k]],
            u_rows.at[pl.ds(k * GATHER_CHUNK, GATHER_CHUNK)], sem))
        copies.append(pltpu.async_copy(
            product_table.at[idx_p.at[k]],
            p_rows.at[pl.ds(k * GATHER_CHUNK, GATHER_CHUNK)], sem))
    for c in copies:
        c.wait()

    wu_c = [wu_v[pl.ds(c * LANES, LANES)] for c in range(EMBED // LANES)]
    wp_c = [wp_v[pl.ds(c * LANES, LANES)] for c in range(EMBED // LANES)]
    b_l = b_v[...]
    lane = lax.iota(jnp.int32, LANES)

    def block_body(g, carry):
        res = b_l
        for i in range(LANES):
            r = g * LANES + i
            t = u_rows[r, pl.ds(0, LANES)] * wu_c[0]
            for c in range(1, EMBED // LANES):
                t = t + u_rows[r, pl.ds(c * LANES, LANES)] * wu_c[c]
            for c in range(EMBED // LANES):
                t = t + p_rows[r, pl.ds(c * LANES, LANES)] * wp_c[c]
            s = jnp.sum(t)
            res = jnp.where(lane == i, s, res)
        y = 1.0 / (1.0 + jnp.exp(-res))
        out_v[pl.ds(g * LANES, LANES)] = y
        return carry

    lax.fori_loop(0, ROWS_PER_W // LANES, block_body, 0)
    pltpu.sync_copy(out_v, out_hbm.at[pl.ds(wid * ROWS_PER_W, ROWS_PER_W)])


def kernel(user_ids, product_ids, user_table, product_table, W, b):
    wu = W[:EMBED, 0]
    wp = W[EMBED:, 0]
    bvec = jnp.broadcast_to(b, (LANES,)).astype(jnp.float32)
    uid2 = user_ids.reshape(BATCH // GATHER_CHUNK, GATHER_CHUNK)
    pid2 = product_ids.reshape(BATCH // GATHER_CHUNK, GATHER_CHUNK)

    mesh = plsc.VectorSubcoreMesh(core_axis_name="c", subcore_axis_name="s")
    run = functools.partial(
        pl.kernel, mesh=mesh,
        compiler_params=pltpu.CompilerParams(
            needs_layout_passes=False, use_tc_tiling_on_sc=False),
        out_type=jax.ShapeDtypeStruct((BATCH,), jnp.float32),
        scratch_types=[
            pltpu.VMEM((NCHUNK, GATHER_CHUNK), jnp.int32),
            pltpu.VMEM((NCHUNK, GATHER_CHUNK), jnp.int32),
            pltpu.VMEM((ROWS_PER_W, EMBED), jnp.float32),
            pltpu.VMEM((ROWS_PER_W, EMBED), jnp.float32),
            pltpu.VMEM((ROWS_PER_W,), jnp.float32),
            pltpu.VMEM((EMBED,), jnp.float32),
            pltpu.VMEM((EMBED,), jnp.float32),
            pltpu.VMEM((LANES,), jnp.float32),
            pltpu.SemaphoreType.DMA,
        ],
    )(_sc_kernel)
    out = run(uid2, pid2, user_table, product_table, wu, wp, bvec)
    return out.reshape(BATCH, 1)


# speedup vs baseline: 5.3098x; 5.3098x over previous
import functools

import jax
import jax.numpy as jnp
from jax import lax
from jax.experimental import pallas as pl
from jax.experimental.pallas import tpu as pltpu
from jax.experimental.pallas import tpu_sc as plsc

NUM_USERS = 1000000
NUM_PRODUCTS = 1000000
EMBED = 64
BATCH = 16384

NC, NS, LANES = 2, 16, 16
NW = NC * NS
ROWS_PER_W = BATCH // NW
GCHUNK = 128
NCHUNK = ROWS_PER_W // GCHUNK

BLK = 8192


def _tc_matvec(ut_ref, pt_ref, wu_ref, wp_ref, vu_ref, vp_ref):
    vu_ref[...] = jnp.sum(ut_ref[...] * wu_ref[...], axis=0)
    vp_ref[...] = jnp.sum(pt_ref[...] * wp_ref[...], axis=0)


def _sc_gather(uid2, pid2, vu_h, vp_h, bvec, out_hbm,
               idx_u, idx_p, g_u, g_p, out_v, b_v, sem):
    wid = lax.axis_index("s") * NC + lax.axis_index("c")
    base_chunk = wid * NCHUNK

    pltpu.sync_copy(uid2.at[pl.ds(base_chunk, NCHUNK)], idx_u)
    pltpu.sync_copy(pid2.at[pl.ds(base_chunk, NCHUNK)], idx_p)
    pltpu.sync_copy(bvec, b_v)

    copies = []
    for k in range(NCHUNK):
        copies.append(pltpu.async_copy(
            vu_h.at[idx_u.at[k]], g_u.at[pl.ds(k * GCHUNK, GCHUNK)], sem))
        copies.append(pltpu.async_copy(
            vp_h.at[idx_p.at[k]], g_p.at[pl.ds(k * GCHUNK, GCHUNK)], sem))
    for c in copies:
        c.wait()

    b_l = b_v[...]

    def block_body(g, carry):
        sl = pl.ds(g * LANES, LANES)
        acc = g_u[sl] + g_p[sl] + b_l
        out_v[sl] = 1.0 / (1.0 + jnp.exp(-acc))
        return carry

    lax.fori_loop(0, ROWS_PER_W // LANES, block_body, 0)
    pltpu.sync_copy(out_v, out_hbm.at[pl.ds(wid * ROWS_PER_W, ROWS_PER_W)])


def kernel(user_ids, product_ids, user_table, product_table, W, b):
    wu = W[:EMBED, :]
    wp = W[EMBED:, :]
    bvec = jnp.broadcast_to(b, (LANES,)).astype(jnp.float32)
    ut = user_table.T
    pt = product_table.T

    nblk = (NUM_USERS + BLK - 1) // BLK
    vu, vp = pl.pallas_call(
        _tc_matvec,
        grid=(nblk,),
        in_specs=[
            pl.BlockSpec((EMBED, BLK), lambda i: (0, i)),
            pl.BlockSpec((EMBED, BLK), lambda i: (0, i)),
            pl.BlockSpec((EMBED, 1), lambda i: (0, 0)),
            pl.BlockSpec((EMBED, 1), lambda i: (0, 0)),
        ],
        out_specs=[
            pl.BlockSpec((BLK,), lambda i: (i,)),
            pl.BlockSpec((BLK,), lambda i: (i,)),
        ],
        out_shape=[
            jax.ShapeDtypeStruct((NUM_USERS,), jnp.float32),
            jax.ShapeDtypeStruct((NUM_PRODUCTS,), jnp.float32),
        ],
    )(ut, pt, wu, wp)

    uid2 = user_ids.reshape(BATCH // GCHUNK, GCHUNK)
    pid2 = product_ids.reshape(BATCH // GCHUNK, GCHUNK)

    mesh = plsc.VectorSubcoreMesh(core_axis_name="c", subcore_axis_name="s")
    run = functools.partial(
        pl.kernel, mesh=mesh,
        compiler_params=pltpu.CompilerParams(
            needs_layout_passes=False, use_tc_tiling_on_sc=False),
        out_type=jax.ShapeDtypeStruct((BATCH,), jnp.float32),
        scratch_types=[
            pltpu.VMEM((NCHUNK, GCHUNK), jnp.int32),
            pltpu.VMEM((NCHUNK, GCHUNK), jnp.int32),
            pltpu.VMEM((ROWS_PER_W,), jnp.float32),
            pltpu.VMEM((ROWS_PER_W,), jnp.float32),
            pltpu.VMEM((ROWS_PER_W,), jnp.float32),
            pltpu.VMEM((LANES,), jnp.float32),
            pltpu.SemaphoreType.DMA,
        ],
    )(_sc_gather)
    out = run(uid2, pid2, vu, vp, bvec)
    return out.reshape(BATCH, 1)


# speedup vs baseline: 6.2371x; 1.1746x over previous
import functools

import jax
import jax.numpy as jnp
from jax import lax
from jax.experimental import pallas as pl
from jax.experimental.pallas import tpu as pltpu
from jax.experimental.pallas import tpu_sc as plsc

NUM_USERS = 1000000
NUM_PRODUCTS = 1000000
EMBED = 64
BATCH = 16384

NC, NS, LANES = 2, 16, 16
NW = NC * NS
ROWS_PER_W = BATCH // NW
GCHUNK = 128
NCHUNK = ROWS_PER_W // GCHUNK

BLK = 16384


def _tc_matvec(ut_ref, pt_ref, wu_ref, wp_ref, vu_ref, vp_ref):
    vu_ref[...] = jnp.sum(ut_ref[...] * wu_ref[...], axis=0)
    vp_ref[...] = jnp.sum(pt_ref[...] * wp_ref[...], axis=0)


def _sc_gather(uid2, pid2, vu_h, vp_h, bvec, out_hbm,
               idx_u, idx_p, g_u, g_p, out_v, b_v, sem):
    wid = lax.axis_index("s") * NC + lax.axis_index("c")
    base_chunk = wid * NCHUNK

    pltpu.sync_copy(uid2.at[pl.ds(base_chunk, NCHUNK)], idx_u)
    pltpu.sync_copy(pid2.at[pl.ds(base_chunk, NCHUNK)], idx_p)
    pltpu.sync_copy(bvec, b_v)

    copies = []
    for k in range(NCHUNK):
        copies.append(pltpu.async_copy(
            vu_h.at[idx_u.at[k]], g_u.at[pl.ds(k * GCHUNK, GCHUNK)], sem))
        copies.append(pltpu.async_copy(
            vp_h.at[idx_p.at[k]], g_p.at[pl.ds(k * GCHUNK, GCHUNK)], sem))
    for c in copies:
        c.wait()

    b_l = b_v[...]

    def block_body(g, carry):
        sl = pl.ds(g * LANES, LANES)
        acc = g_u[sl] + g_p[sl] + b_l
        out_v[sl] = 1.0 / (1.0 + jnp.exp(-acc))
        return carry

    lax.fori_loop(0, ROWS_PER_W // LANES, block_body, 0)
    pltpu.sync_copy(out_v, out_hbm.at[pl.ds(wid * ROWS_PER_W, ROWS_PER_W)])


def kernel(user_ids, product_ids, user_table, product_table, W, b):
    wu = W[:EMBED, :]
    wp = W[EMBED:, :]
    bvec = jnp.broadcast_to(b, (LANES,)).astype(jnp.float32)
    ut = user_table.T
    pt = product_table.T

    nblk = (NUM_USERS + BLK - 1) // BLK
    vu, vp = pl.pallas_call(
        _tc_matvec,
        grid=(nblk,),
        in_specs=[
            pl.BlockSpec((EMBED, BLK), lambda i: (0, i)),
            pl.BlockSpec((EMBED, BLK), lambda i: (0, i)),
            pl.BlockSpec((EMBED, 1), lambda i: (0, 0)),
            pl.BlockSpec((EMBED, 1), lambda i: (0, 0)),
        ],
        out_specs=[
            pl.BlockSpec((BLK,), lambda i: (i,)),
            pl.BlockSpec((BLK,), lambda i: (i,)),
        ],
        out_shape=[
            jax.ShapeDtypeStruct((NUM_USERS,), jnp.float32),
            jax.ShapeDtypeStruct((NUM_PRODUCTS,), jnp.float32),
        ],
    )(ut, pt, wu, wp)

    uid2 = user_ids.reshape(BATCH // GCHUNK, GCHUNK)
    pid2 = product_ids.reshape(BATCH // GCHUNK, GCHUNK)

    mesh = plsc.VectorSubcoreMesh(core_axis_name="c", subcore_axis_name="s")
    run = functools.partial(
        pl.kernel, mesh=mesh,
        compiler_params=pltpu.CompilerParams(
            needs_layout_passes=False, use_tc_tiling_on_sc=False),
        out_type=jax.ShapeDtypeStruct((BATCH,), jnp.float32),
        scratch_types=[
            pltpu.VMEM((NCHUNK, GCHUNK), jnp.int32),
            pltpu.VMEM((NCHUNK, GCHUNK), jnp.int32),
            pltpu.VMEM((ROWS_PER_W,), jnp.float32),
            pltpu.VMEM((ROWS_PER_W,), jnp.float32),
            pltpu.VMEM((ROWS_PER_W,), jnp.float32),
            pltpu.VMEM((LANES,), jnp.float32),
            pltpu.SemaphoreType.DMA,
        ],
    )(_sc_gather)
    out = run(uid2, pid2, vu, vp, bvec)
    return out.reshape(BATCH, 1)
